# Initial kernel scaffold; baseline (speedup 1.0000x reference)
import jax
import jax.numpy as jnp
from jax import lax
from jax.experimental import pallas as pl
from jax.experimental.pallas import tpu as pltpu
from jax.experimental.pallas import tpu_sc as plsc

N = 10000
E = 160000
H = 4
F = 256
ND = 4000
OUT = 256
B = 8192
NEG = 0.2

NC = 2
NS = 16
L = 16

NPAD = 10240
EPT = E // NS
CHA = 400
CHS = 400
MCB = 16
PPW = B // (NC * NS)
CHC = 64

_mesh = plsc.VectorSubcoreMesh(core_axis_name="c", subcore_axis_name="s")


def _tc1_body(x_ref, wg_ref, asrc_ref, adst_ref, z_ref, esed_ref):
    x = x_ref[...]
    wg = wg_ref[...]
    z = jnp.dot(x, wg, preferred_element_type=jnp.float32)
    z_ref[...] = z
    cols = []
    for hp in range(2):
        for hh in range(2):
            h = 2 * hp + hh
            a = asrc_ref[h, :].reshape(F, 1)
            cols.append(jnp.dot(z[:, h * F:(h + 1) * F], a,
                                preferred_element_type=jnp.float32))
        for hh in range(2):
            h = 2 * hp + hh
            a = adst_ref[h, :].reshape(F, 1)
            cols.append(jnp.dot(z[:, h * F:(h + 1) * F], a,
                                preferred_element_type=jnp.float32))
    esed_ref[...] = jnp.concatenate(cols, axis=1)


def _tc1(node_feat, Wg, a_src, a_dst):
    return pl.pallas_call(
        _tc1_body,
        grid=(10,),
        in_specs=[
            pl.BlockSpec((1000, F), lambda i: (i, 0)),
            pl.BlockSpec((F, H * F), lambda i: (0, 0)),
            pl.BlockSpec((H, F), lambda i: (0, 0)),
            pl.BlockSpec((H, F), lambda i: (0, 0)),
        ],
        out_specs=[
            pl.BlockSpec((1000, H * F), lambda i: (i, 0)),
            pl.BlockSpec((1000, 8), lambda i: (i, 0)),
        ],
        out_shape=[
            jax.ShapeDtypeStruct((N, H * F), jnp.float32),
            jax.ShapeDtypeStruct((N, 8), jnp.float32),
        ],
    )(node_feat, Wg, a_src, a_dst)


def _sca_body(esed_hbm, src_hbm, dst_hbm, expf_hbm, denf_hbm,
              tab_v, acc_v, srcb, dstb, expb, res_v, red_v, shared_p):
    cid = lax.axis_index("c")
    sid = lax.axis_index("s")
    ebase = sid * EPT

    pltpu.sync_copy(esed_hbm.at[cid], tab_v)

    def _zero(i, _):
        acc_v[pl.ds(i * L, L)] = jnp.zeros((L,), jnp.float32)
        return 0
    lax.fori_loop(0, (2 * NPAD) // L, _zero, 0)

    def _chunk(ch, _):
        off = ebase + ch * CHA
        pltpu.sync_copy(src_hbm.at[pl.ds(off, CHA)], srcb)
        pltpu.sync_copy(dst_hbm.at[pl.ds(off, CHA)], dstb)

        def _group(g, _):
            sv = srcb[pl.ds(g * L, L)]
            dv = dstb[pl.ds(g * L, L)]
            for hh in range(2):
                es = plsc.load_gather(tab_v, [sv, jnp.full((L,), hh, jnp.int32)])
                ed = plsc.load_gather(tab_v, [dv, jnp.full((L,), 2 + hh, jnp.int32)])
                e = es + ed
                e = jnp.where(e >= 0, e, NEG * e)
                p = jnp.exp(e)
                plsc.addupdate_scatter(acc_v, [dv * 2 + hh], p)
                expb[hh, pl.ds(g * L, L)] = p
            return 0
        lax.fori_loop(0, CHA // L, _group, 0)
        for hh in range(2):
            pltpu.sync_copy(expb.at[hh], expf_hbm.at[cid, pl.ds(hh * E + off, CHA)])
        return 0
    lax.fori_loop(0, EPT // CHA, _chunk, 0)

    pltpu.sync_copy(acc_v, shared_p.at[sid])
    plsc.subcore_barrier()
    words = (2 * NPAD) // NS
    woff = sid * words
    pltpu.sync_copy(shared_p.at[:, pl.ds(woff, words)], red_v)

    def _red(j, _):
        s = jnp.zeros((L,), jnp.float32)
        for t in range(NS):
            s = s + red_v[t, pl.ds(j * L, L)]
        res_v[pl.ds(j * L, L)] = s
        return 0
    lax.fori_loop(0, words // L, _red, 0)
    pltpu.sync_copy(res_v, denf_hbm.at[cid, pl.ds(woff, words)])


def _sca(esed_slabs, src, dst):
    words = (2 * NPAD) // NS
    f = pl.kernel(
        _sca_body,
        out_type=(
            jax.ShapeDtypeStruct((2, 2 * E), jnp.float32),
            jax.ShapeDtypeStruct((2, 2 * NPAD), jnp.float32),
        ),
        mesh=_mesh,
        scratch_types=[
            pltpu.VMEM((N, 4), jnp.float32),
            pltpu.VMEM((2 * NPAD,), jnp.float32),
            pltpu.VMEM((CHA,), jnp.int32),
            pltpu.VMEM((CHA,), jnp.int32),
            pltpu.VMEM((2, CHA), jnp.float32),
            pltpu.VMEM((words,), jnp.float32),
            pltpu.VMEM((NS, words), jnp.float32),
            pltpu.VMEM_SHARED((NS, 2 * NPAD), jnp.float32),
        ],
    )
    return f(esed_slabs, src, dst)


ZROWS_PER_MC = MCB * H
FH = F // NC


def _scb_body(zr_hbm, src_hbm, dst_hbm, expf_hbm, denf_hbm, hm_hbm,
              den_v, srcb, dstb, expsv, al_v, idxg, rows_v, msg_v, dsti,
              zb, sem, hacc_sh):
    cid = lax.axis_index("c")
    sid = lax.axis_index("s")

    for hp in range(2):
        pltpu.sync_copy(denf_hbm.at[hp], den_v.at[pl.ds(hp * 2 * NPAD, 2 * NPAD)])

    def _zb(i, _):
        zb[pl.ds(i * L, L)] = jnp.zeros((L,), jnp.float32)
        return 0
    lax.fori_loop(0, (125 * FH) // L, _zb, 0)
    zb2 = zb.reshape(125, FH)
    for r in range(5):
        pltpu.sync_copy(zb2, hacc_sh.at[pl.ds(sid * 625 + r * 125, 125)])
    plsc.subcore_barrier()

    ebase = sid * EPT

    def _super(sc, _):
        soff = ebase + sc * CHS
        pltpu.sync_copy(src_hbm.at[pl.ds(soff, CHS)], srcb)
        pltpu.sync_copy(dst_hbm.at[pl.ds(soff, CHS)], dstb)
        for hp in range(2):
            for hh in range(2):
                pltpu.sync_copy(expf_hbm.at[hp, pl.ds(hh * E + soff, CHS)],
                                expsv.at[2 * hp + hh])

        def _micro(mc, _):
            mb = mc * MCB
            sv = srcb[pl.ds(mb, L)]
            dv = dstb[pl.ds(mb, L)]
            dsti[...] = dv
            for h in range(H):
                idxg[pl.ds(h * MCB, L)] = sv * 8 + (h * 2 + cid)
            for hp in range(2):
                for hh in range(2):
                    h = 2 * hp + hh
                    ev = expsv[h, pl.ds(mb, L)]
                    dd = plsc.load_gather(den_v, [hp * 2 * NPAD + dv * 2 + hh])
                    al_v[h, :] = (0.25 * ev) / dd
            pltpu.async_copy(zr_hbm.at[idxg], rows_v, sem).wait()

            def _edge(k, _):
                a0 = al_v[0, k]
                a1 = al_v[1, k]
                a2 = al_v[2, k]
                a3 = al_v[3, k]
                for j in range(FH // L):
                    m = a0 * rows_v[k, pl.ds(j * L, L)]
                    m = m + a1 * rows_v[MCB + k, pl.ds(j * L, L)]
                    m = m + a2 * rows_v[2 * MCB + k, pl.ds(j * L, L)]
                    m = m + a3 * rows_v[3 * MCB + k, pl.ds(j * L, L)]
                    msg_v[k, pl.ds(j * L, L)] = m
                return 0
            lax.fori_loop(0, MCB, _edge, 0)
            pltpu.sync_copy(msg_v, hacc_sh.at[dsti], add=True)
            return 0
        lax.fori_loop(0, CHS // MCB, _micro, 0)
        return 0
    lax.fori_loop(0, EPT // CHS, _super, 0)

    plsc.subcore_barrier()
    pltpu.sync_copy(hacc_sh.at[pl.ds(sid * 625, 625)],
                    hm_hbm.at[cid, pl.ds(sid * 625, 625)])


def _scb(zr, src, dst, expf, denf):
    f = pl.kernel(
        _scb_body,
        out_type=jax.ShapeDtypeStruct((2, N, FH), jnp.float32),
        mesh=_mesh,
        scratch_types=[
            pltpu.VMEM((2 * 2 * NPAD,), jnp.float32),
            pltpu.VMEM((CHS,), jnp.int32),
            pltpu.VMEM((CHS,), jnp.int32),
            pltpu.VMEM((H, CHS), jnp.float32),
            pltpu.VMEM((H, MCB), jnp.float32),
            pltpu.VMEM((ZROWS_PER_MC,), jnp.int32),
            pltpu.VMEM((ZROWS_PER_MC, FH), jnp.float32),
            pltpu.VMEM((MCB, FH), jnp.float32),
            pltpu.VMEM((MCB,), jnp.int32),
            pltpu.VMEM((125 * FH,), jnp.float32),
            pltpu.SemaphoreType.DMA,
            pltpu.VMEM_SHARED((N, FH), jnp.float32),
        ],
    )
    return f(zr, src, dst, expf, denf)


def _tc2_body(hm0_ref, hm1_ref, sim_ref, w_ref, b_ref, wdec_ref, h_ref, g_ref):
    w = w_ref[0]
    acc = jnp.dot(hm0_ref[...], w[:FH, :], preferred_element_type=jnp.float32)
    acc += jnp.dot(hm1_ref[...], w[FH:2 * FH, :], preferred_element_type=jnp.float32)
    acc += jnp.dot(sim_ref[...], w[2 * FH:, :], preferred_element_type=jnp.float32)
    acc += b_ref[...]
    h = jnp.where(acc > 0, acc, jnp.exp(jnp.minimum(acc, 0.0)) - 1.0)
    h_ref[...] = h
    g_ref[...] = jnp.dot(h, wdec_ref[...], preferred_element_type=jnp.float32)


def _tc2(hm0, hm1, sim, w_stack, b_stack, W_dec):
    sel3 = lambda i: (jnp.minimum(i // 4, 1), 0, 0)
    sel2 = lambda i: (jnp.minimum(i // 4, 1), 0)
    return pl.pallas_call(
        _tc2_body,
        grid=(10,),
        in_specs=[
            pl.BlockSpec((1000, FH), lambda i: (i, 0)),
            pl.BlockSpec((1000, FH), lambda i: (i, 0)),
            pl.BlockSpec((1000, FH), lambda i: (i, 0)),
            pl.BlockSpec((1, 3 * FH, OUT), sel3),
            pl.BlockSpec((1, OUT), sel2),
            pl.BlockSpec((OUT, OUT), lambda i: (0, 0)),
        ],
        out_specs=[
            pl.BlockSpec((1000, OUT), lambda i: (i, 0)),
            pl.BlockSpec((1000, OUT), lambda i: (i, 0)),
        ],
        out_shape=[
            jax.ShapeDtypeStruct((N, OUT), jnp.float32),
            jax.ShapeDtypeStruct((N, OUT), jnp.float32),
        ],
    )(hm0, hm1, sim, w_stack, b_stack, W_dec)


def _scc_body(g_hbm, h_hbm, dis_hbm, mir_hbm, out_hbm,
              idxd, idxm, gv, hv, sbuf, obuf, sem):
    cid = lax.axis_index("c")
    sid = lax.axis_index("s")
    wid = sid * NC + cid
    wbase = wid * PPW

    def _chunk(c, _):
        base = wbase + c * CHC
        pltpu.sync_copy(dis_hbm.at[pl.ds(base, CHC)], idxd)
        pltpu.sync_copy(mir_hbm.at[pl.ds(base, CHC)], idxm)
        pltpu.async_copy(g_hbm.at[idxd], gv, sem).wait()
        pltpu.async_copy(h_hbm.at[idxm], hv, sem).wait()

        def _pair(k, _):
            s = jnp.zeros((L,), jnp.float32)
            for j in range(OUT // L):
                s = s + gv[k, pl.ds(j * L, L)] * hv[k, pl.ds(j * L, L)]
            sbuf[k] = jnp.sum(s)
            return 0
        lax.fori_loop(0, CHC, _pair, 0)

        def _sig(g, _):
            v = sbuf[pl.ds(g * L, L)]
            obuf[pl.ds(g * L, L)] = 1.0 / (1.0 + jnp.exp(-v))
            return 0
        lax.fori_loop(0, CHC // L, _sig, 0)
        pltpu.sync_copy(obuf, out_hbm.at[pl.ds(base, CHC)])
        return 0
    lax.fori_loop(0, PPW // CHC, _chunk, 0)


def _scc(G, h, diseases, mrnas):
    f = pl.kernel(
        _scc_body,
        out_type=jax.ShapeDtypeStruct((B,), jnp.float32),
        mesh=_mesh,
        scratch_types=[
            pltpu.VMEM((CHC,), jnp.int32),
            pltpu.VMEM((CHC,), jnp.int32),
            pltpu.VMEM((CHC, OUT), jnp.float32),
            pltpu.VMEM((CHC, OUT), jnp.float32),
            pltpu.VMEM((CHC,), jnp.float32),
            pltpu.VMEM((CHC,), jnp.float32),
            pltpu.SemaphoreType.DMA,
        ],
    )
    return f(G, h, diseases, mrnas)


def kernel(node_feat, d_sim, m_sim, edge_index, diseases, mrnas,
           Wg, a_src, a_dst, m_fc_W, m_fc_b, d_fc_W, d_fc_b, W_dec):
    src = edge_index[0].astype(jnp.int32)
    dst = edge_index[1].astype(jnp.int32)

    z, esed = _tc1(node_feat, Wg, a_src, a_dst)
    esed_slabs = esed.reshape(N, 2, 4).transpose(1, 0, 2)
    zr = z.reshape(N * 8, FH)

    expf, denf = _sca(esed_slabs, src, dst)
    hm = _scb(zr, src, dst, expf, denf)

    sim = jnp.concatenate([d_sim[:ND], m_sim[ND:]], axis=0)
    w_stack = jnp.stack([d_fc_W, m_fc_W])
    b_stack = jnp.stack([d_fc_b, m_fc_b])

    h, G = _tc2(hm[0], hm[1], sim, w_stack, b_stack, W_dec)
    return _scc(G, h, diseases.astype(jnp.int32), mrnas.astype(jnp.int32))


# speedup vs baseline: 9.9746x; 9.9746x over previous
import jax
import jax.numpy as jnp
from jax import lax
from jax.experimental import pallas as pl
from jax.experimental.pallas import tpu as pltpu
from jax.experimental.pallas import tpu_sc as plsc

N = 10000
E = 160000
H = 4
F = 256
ND = 4000
OUT = 256
B = 8192
NEG = 0.2

NC = 2
NS = 16
L = 16
NW = NC * NS

NPAD = 10240
FH = F // NC

CHA = 128
NCH_A = E // CHA
BCB = 32
NBL_B = E // BCB
PPW = B // NW
CHC = 64

_mesh = plsc.VectorSubcoreMesh(core_axis_name="c", subcore_axis_name="s")


def _tc1_body(x_ref, wg_ref, asrc_ref, adst_ref, z_ref, esed_ref):
    x = x_ref[...]
    wg = wg_ref[...]
    z = jnp.dot(x, wg, preferred_element_type=jnp.float32)
    z_ref[...] = z
    cols = []
    for aref in (asrc_ref, adst_ref):
        for h in range(H):
            a = aref[pl.ds(h, 1), :]
            cols.append(jnp.sum(z[:, h * F:(h + 1) * F] * a, axis=1,
                                keepdims=True))
    esed_ref[...] = jnp.concatenate(cols, axis=1)


def _tc1(node_feat, Wg, a_src, a_dst):
    return pl.pallas_call(
        _tc1_body,
        grid=(10,),
        in_specs=[
            pl.BlockSpec((1000, F), lambda i: (i, 0)),
            pl.BlockSpec((F, H * F), lambda i: (0, 0)),
            pl.BlockSpec((H, F), lambda i: (0, 0)),
            pl.BlockSpec((H, F), lambda i: (0, 0)),
        ],
        out_specs=[
            pl.BlockSpec((1000, H * F), lambda i: (i, 0)),
            pl.BlockSpec((1000, 8), lambda i: (i, 0)),
        ],
        out_shape=[
            jax.ShapeDtypeStruct((N, H * F), jnp.float32),
            jax.ShapeDtypeStruct((N, 8), jnp.float32),
        ],
    )(node_feat, Wg, a_src, a_dst)


def _sca_body(esed_hbm, src_hbm, dst_hbm, expf_hbm, denp_hbm,
              srcb, dstb, isrc, idst, idxd, esv, edv, pb, zba, gsem, den_sh):
    cid = lax.axis_index("c")
    sid = lax.axis_index("s")
    wid = sid * NC + cid

    def _zero(i, _):
        zba[pl.ds(i * L, L)] = jnp.zeros((L,), jnp.float32)
        return 0
    lax.fori_loop(0, (4 * NPAD // NS) // L, _zero, 0)
    pltpu.sync_copy(zba, den_sh.at[pl.ds(sid * (4 * NPAD // NS),
                                         4 * NPAD // NS)])
    plsc.subcore_barrier()

    def _do_chunk(cno):
        eoff = cno * CHA
        pltpu.sync_copy(src_hbm.at[pl.ds(eoff, CHA)], srcb)
        pltpu.sync_copy(dst_hbm.at[pl.ds(eoff, CHA)], dstb)
        def _bidx(g, _):
            sv = srcb[pl.ds(g * L, L)]
            dv = dstb[pl.ds(g * L, L)]
            for h in range(H):
                isrc[h, pl.ds(g * L, L)] = sv * 8 + h
                idst[h, pl.ds(g * L, L)] = dv * 8 + (4 + h)
                idxd[h, pl.ds(g * L, L)] = dv * 4 + h
            return 0
        lax.fori_loop(0, CHA // L, _bidx, 0)
        cps = []
        for h in range(H):
            cps.append(pltpu.async_copy(esed_hbm.at[isrc.at[h]], esv.at[h], gsem))
            cps.append(pltpu.async_copy(esed_hbm.at[idst.at[h]], edv.at[h], gsem))
        for cp in cps:
            cp.wait()

        def _grp(g, _):
            for h in range(H):
                e = esv[h, pl.ds(g * L, L)] + edv[h, pl.ds(g * L, L)]
                e = jnp.where(e >= 0, e, NEG * e)
                pb[h, pl.ds(g * L, L)] = jnp.exp(e)
            return 0
        lax.fori_loop(0, CHA // L, _grp, 0)
        for h in range(H):
            pltpu.sync_copy(pb.at[h], expf_hbm.at[pl.ds(h * E + eoff, CHA)])
            pltpu.sync_copy(pb.at[h], den_sh.at[idxd.at[h]], add=True)

    def _chunk(j, _):
        _do_chunk(wid + NW * j)
        return 0
    nfull = NCH_A // NW
    lax.fori_loop(0, nfull, _chunk, 0)
    @pl.when(wid < NCH_A - nfull * NW)
    def _():
        _do_chunk(nfull * NW + wid)

    plsc.subcore_barrier()
    w = 4 * NPAD // NS
    pltpu.sync_copy(den_sh.at[pl.ds(sid * w, w)],
                    denp_hbm.at[pl.ds(cid * 4 * NPAD + sid * w, w)])


def _sca(esed_flat, src, dst):
    w = 4 * NPAD // NS
    f = pl.kernel(
        _sca_body,
        out_type=(
            jax.ShapeDtypeStruct((H * E,), jnp.float32),
            jax.ShapeDtypeStruct((NC * 4 * NPAD,), jnp.float32),
        ),
        mesh=_mesh,
        scratch_types=[
            pltpu.VMEM((CHA,), jnp.int32),
            pltpu.VMEM((CHA,), jnp.int32),
            pltpu.VMEM((H, CHA), jnp.int32),
            pltpu.VMEM((H, CHA), jnp.int32),
            pltpu.VMEM((H, CHA), jnp.int32),
            pltpu.VMEM((H, CHA), jnp.float32),
            pltpu.VMEM((H, CHA), jnp.float32),
            pltpu.VMEM((H, CHA), jnp.float32),
            pltpu.VMEM((w,), jnp.float32),
            pltpu.SemaphoreType.DMA,
            pltpu.VMEM_SHARED((4 * NPAD,), jnp.float32),
        ],
    )
    return f(esed_flat, src, dst)


def _sca2_body(expf_hbm, denp_hbm, dst_hbm, alpha_hbm,
               dstb, idxd, pv, d0, d1, gsem):
    cid = lax.axis_index("c")
    sid = lax.axis_index("s")
    wid = sid * NC + cid

    def _do_chunk(cno):
        eoff = cno * CHA
        pltpu.sync_copy(dst_hbm.at[pl.ds(eoff, CHA)], dstb)
        def _bidx(g, _):
            dv = dstb[pl.ds(g * L, L)]
            for h in range(H):
                idxd[h, pl.ds(g * L, L)] = dv * 4 + h
            return 0
        lax.fori_loop(0, CHA // L, _bidx, 0)
        cps = []
        for h in range(H):
            cps.append(pltpu.async_copy(
                expf_hbm.at[pl.ds(h * E + eoff, CHA)], pv.at[h], gsem))
            cps.append(pltpu.async_copy(denp_hbm.at[idxd.at[h]], d0.at[h], gsem))
        for cp in cps:
            cp.wait()
        def _bidx2(g, _):
            for h in range(H):
                idxd[h, pl.ds(g * L, L)] = idxd[h, pl.ds(g * L, L)] + 4 * NPAD
            return 0
        lax.fori_loop(0, CHA // L, _bidx2, 0)
        cps = [pltpu.async_copy(denp_hbm.at[idxd.at[h]], d1.at[h], gsem)
               for h in range(H)]
        for cp in cps:
            cp.wait()
        def _grp(g, _):
            for h in range(H):
                den = d0[h, pl.ds(g * L, L)] + d1[h, pl.ds(g * L, L)]
                pv[h, pl.ds(g * L, L)] = 0.25 * pv[h, pl.ds(g * L, L)] / den
            return 0
        lax.fori_loop(0, CHA // L, _grp, 0)
        for h in range(H):
            pltpu.sync_copy(pv.at[h], alpha_hbm.at[pl.ds(h * E + eoff, CHA)])

    def _chunk(j, _):
        _do_chunk(wid + NW * j)
        return 0
    nfull = NCH_A // NW
    lax.fori_loop(0, nfull, _chunk, 0)
    @pl.when(wid < NCH_A - nfull * NW)
    def _():
        _do_chunk(nfull * NW + wid)


def _sca2(expf, denp, dst):
    f = pl.kernel(
        _sca2_body,
        out_type=jax.ShapeDtypeStruct((H * E,), jnp.float32),
        mesh=_mesh,
        scratch_types=[
            pltpu.VMEM((CHA,), jnp.int32),
            pltpu.VMEM((H, CHA), jnp.int32),
            pltpu.VMEM((H, CHA), jnp.float32),
            pltpu.VMEM((H, CHA), jnp.float32),
            pltpu.VMEM((H, CHA), jnp.float32),
            pltpu.SemaphoreType.DMA,
        ],
    )
    return f(expf, denp, dst)


def _scb_body(zr_hbm, src_hbm, dst_hbm, arep_hbm, hm_hbm,
              srcb, dsti, idxg, rows_v, arows, msg_v, zb, gsem, hacc_sh):
    cid = lax.axis_index("c")
    sid = lax.axis_index("s")

    def _zb(i, _):
        for j in range(FH // L):
            zb[i, pl.ds(j * L, L)] = jnp.zeros((L,), jnp.float32)
        return 0
    lax.fori_loop(0, 128, _zb, 0)
    for r in range(5):
        pltpu.sync_copy(zb, hacc_sh.at[pl.ds(sid * 640 + r * 128, 128)])
    plsc.subcore_barrier()

    def _block(jb, _):
        bb = sid + NS * jb
        boff = bb * BCB
        pltpu.sync_copy(src_hbm.at[pl.ds(boff, BCB)], srcb)
        pltpu.sync_copy(dst_hbm.at[pl.ds(boff, BCB)], dsti)
        cpa = pltpu.async_copy(
            arep_hbm.at[pl.ds(bb * (BCB // 2), BCB // 2)], arows, gsem)
        def _bidx(g, _):
            sv = srcb[pl.ds(g * L, L)]
            for h in range(H):
                idxg[pl.ds(h * BCB + g * L, L)] = sv * 8 + (h * 2 + cid)
            return 0
        lax.fori_loop(0, BCB // L, _bidx, 0)
        cpr = pltpu.async_copy(zr_hbm.at[idxg], rows_v, gsem)
        cpa.wait()
        cpr.wait()

        def _edge(k, _):
            r2 = k // 2
            lo = (k % 2) * 64
            for j in range(FH // L):
                m = (arows[r2, pl.ds(lo, L)] *
                     rows_v[k, pl.ds(j * L, L)])
                m = m + (arows[r2, pl.ds(lo + 16, L)] *
                         rows_v[BCB + k, pl.ds(j * L, L)])
                m = m + (arows[r2, pl.ds(lo + 32, L)] *
                         rows_v[2 * BCB + k, pl.ds(j * L, L)])
                m = m + (arows[r2, pl.ds(lo + 48, L)] *
                         rows_v[3 * BCB + k, pl.ds(j * L, L)])
                msg_v[k, pl.ds(j * L, L)] = m
            return 0
        lax.fori_loop(0, BCB, _edge, 0)
        pltpu.sync_copy(msg_v, hacc_sh.at[dsti], add=True)
        return 0
    lax.fori_loop(0, NBL_B // NS, _block, 0)
    @pl.when(sid < NBL_B - (NBL_B // NS) * NS)
    def _():
        _ = _block(NBL_B // NS, 0)

    plsc.subcore_barrier()
    pltpu.sync_copy(hacc_sh.at[pl.ds(sid * 640, 640)],
                    hm_hbm.at[cid, pl.ds(sid * 640, 640)])


def _scb(zr, src, dst, arep):
    f = pl.kernel(
        _scb_body,
        out_type=jax.ShapeDtypeStruct((NC, NPAD, FH), jnp.float32),
        mesh=_mesh,
        scratch_types=[
            pltpu.VMEM((BCB,), jnp.int32),
            pltpu.VMEM((BCB,), jnp.int32),
            pltpu.VMEM((H * BCB,), jnp.int32),
            pltpu.VMEM((H * BCB, FH), jnp.float32),
            pltpu.VMEM((BCB // 2, FH), jnp.float32),
            pltpu.VMEM((BCB, FH), jnp.float32),
            pltpu.VMEM((128, FH), jnp.float32),
            pltpu.SemaphoreType.DMA,
            pltpu.VMEM_SHARED((NPAD, FH), jnp.float32),
        ],
    )
    return f(zr, src, dst, arep)


def _tc2_body(hm0_ref, hm1_ref, sim_ref, w_ref, b_ref, wdec_ref, h_ref, g_ref):
    w = w_ref[0]
    acc = jnp.dot(hm0_ref[...], w[:FH, :], preferred_element_type=jnp.float32)
    acc += jnp.dot(hm1_ref[...], w[FH:2 * FH, :], preferred_element_type=jnp.float32)
    acc += jnp.dot(sim_ref[...], w[2 * FH:, :], preferred_element_type=jnp.float32)
    acc += b_ref[0][0:1, :]
    h = jnp.where(acc > 0, acc, jnp.exp(jnp.minimum(acc, 0.0)) - 1.0)
    h_ref[...] = h
    g_ref[...] = jnp.dot(h, wdec_ref[...], preferred_element_type=jnp.float32)


def _tc2(hm0, hm1, sim, w_stack, b_stack, W_dec):
    sel3 = lambda i: (lax.min(i // 4, 1), 0, 0)
    return pl.pallas_call(
        _tc2_body,
        grid=(10,),
        in_specs=[
            pl.BlockSpec((1000, FH), lambda i: (i, 0)),
            pl.BlockSpec((1000, FH), lambda i: (i, 0)),
            pl.BlockSpec((1000, FH), lambda i: (i, 0)),
            pl.BlockSpec((1, 3 * FH, OUT), sel3),
            pl.BlockSpec((1, 8, OUT), sel3),
            pl.BlockSpec((OUT, OUT), lambda i: (0, 0)),
        ],
        out_specs=[
            pl.BlockSpec((1000, OUT), lambda i: (i, 0)),
            pl.BlockSpec((1000, OUT), lambda i: (i, 0)),
        ],
        out_shape=[
            jax.ShapeDtypeStruct((N, OUT), jnp.float32),
            jax.ShapeDtypeStruct((N, OUT), jnp.float32),
        ],
    )(hm0, hm1, sim, w_stack, b_stack, W_dec)


def _scc_body(g_hbm, h_hbm, dis_hbm, mir_hbm, gd_hbm, hm_hbm,
              idxd, idxm, gv, hv, gsem):
    cid = lax.axis_index("c")
    sid = lax.axis_index("s")
    wid = sid * NC + cid
    wbase = wid * PPW

    def _chunk(c, _):
        base = wbase + c * CHC
        pltpu.sync_copy(dis_hbm.at[pl.ds(base, CHC)], idxd)
        pltpu.sync_copy(mir_hbm.at[pl.ds(base, CHC)], idxm)
        cg = pltpu.async_copy(g_hbm.at[idxd], gv, gsem)
        ch = pltpu.async_copy(h_hbm.at[idxm], hv, gsem)
        cg.wait()
        ch.wait()
        pltpu.sync_copy(gv, gd_hbm.at[pl.ds(base, CHC)])
        pltpu.sync_copy(hv, hm_hbm.at[pl.ds(base, CHC)])
        return 0
    lax.fori_loop(0, PPW // CHC, _chunk, 0)


def _scc(G, h, diseases, mrnas):
    f = pl.kernel(
        _scc_body,
        out_type=(
            jax.ShapeDtypeStruct((B, OUT), jnp.float32),
            jax.ShapeDtypeStruct((B, OUT), jnp.float32),
        ),
        mesh=_mesh,
        scratch_types=[
            pltpu.VMEM((CHC,), jnp.int32),
            pltpu.VMEM((CHC,), jnp.int32),
            pltpu.VMEM((CHC, OUT), jnp.float32),
            pltpu.VMEM((CHC, OUT), jnp.float32),
            pltpu.SemaphoreType.DMA,
        ],
    )
    return f(G, h, diseases, mrnas)


def _tc3_body(gd_ref, hm_ref, o_ref):
    s = jnp.sum(gd_ref[...] * hm_ref[...], axis=1)
    o_ref[...] = 1.0 / (1.0 + jnp.exp(-s))


def _tc3(Gd, Hm):
    return pl.pallas_call(
        _tc3_body,
        grid=(8,),
        in_specs=[
            pl.BlockSpec((1024, OUT), lambda i: (i, 0)),
            pl.BlockSpec((1024, OUT), lambda i: (i, 0)),
        ],
        out_specs=pl.BlockSpec((1024,), lambda i: (i,)),
        out_shape=jax.ShapeDtypeStruct((B,), jnp.float32),
    )(Gd, Hm)


def kernel(node_feat, d_sim, m_sim, edge_index, diseases, mrnas,
           Wg, a_src, a_dst, m_fc_W, m_fc_b, d_fc_W, d_fc_b, W_dec):
    src = edge_index[0].astype(jnp.int32)
    dst = edge_index[1].astype(jnp.int32)

    z, esed = _tc1(node_feat, Wg, a_src, a_dst)
    zr = z.reshape(N * 8, FH)

    expf, denp = _sca(esed.reshape(-1), src, dst)
    alphaE = _sca2(expf, denp, dst)

    arep = jnp.broadcast_to(
        alphaE.reshape(H, E).T.reshape(E // 2, 8, 1), (E // 2, 8, L)
    ).reshape(E // 2, 8 * L)

    hm = _scb(zr, src, dst, arep)

    sim = jnp.concatenate([d_sim[:ND], m_sim[ND:]], axis=0)
    w_stack = jnp.stack([d_fc_W, m_fc_W])
    b_stack = jnp.broadcast_to(jnp.stack([d_fc_b, m_fc_b])[:, None, :],
                               (2, 8, OUT))

    h, G = _tc2(hm[0, :N], hm[1, :N], sim, w_stack, b_stack, W_dec)
    Gd, Hm = _scc(G, h, diseases.astype(jnp.int32), mrnas.astype(jnp.int32))
    return _tc3(Gd, Hm)


# speedup vs baseline: 16.0175x; 1.6058x over previous
import jax
import jax.numpy as jnp
from jax import lax
from jax.experimental import pallas as pl
from jax.experimental.pallas import tpu as pltpu
from jax.experimental.pallas import tpu_sc as plsc

N = 10000
E = 160000
H = 4
F = 256
ND = 4000
OUT = 256
B = 8192
NEG = 0.2

NC = 2
NS = 16
L = 16
NW = NC * NS

NPAD = 10240
FH = F // NC

CHA = 128
NCH_A = E // CHA
BCB = 32
NBL_B = E // BCB
PPW = B // NW
CHC = 64

_mesh = plsc.VectorSubcoreMesh(core_axis_name="c", subcore_axis_name="s")


def _tc1_body(x_ref, wg_ref, asrc_ref, adst_ref, z_ref, esed_ref):
    x = x_ref[...]
    wg = wg_ref[...]
    z = jnp.dot(x, wg, preferred_element_type=jnp.float32)
    z_ref[...] = z
    cols = []
    for aref in (asrc_ref, adst_ref):
        for h in range(H):
            a = aref[pl.ds(h, 1), :]
            cols.append(jnp.sum(z[:, h * F:(h + 1) * F] * a, axis=1,
                                keepdims=True))
    esed_ref[...] = jnp.concatenate(cols, axis=1)


def _tc1(node_feat, Wg, a_src, a_dst):
    return pl.pallas_call(
        _tc1_body,
        grid=(10,),
        in_specs=[
            pl.BlockSpec((1000, F), lambda i: (i, 0)),
            pl.BlockSpec((F, H * F), lambda i: (0, 0)),
            pl.BlockSpec((H, F), lambda i: (0, 0)),
            pl.BlockSpec((H, F), lambda i: (0, 0)),
        ],
        out_specs=[
            pl.BlockSpec((1000, H * F), lambda i: (i, 0)),
            pl.BlockSpec((1000, 8), lambda i: (i, 0)),
        ],
        out_shape=[
            jax.ShapeDtypeStruct((N, H * F), jnp.float32),
            jax.ShapeDtypeStruct((N, 8), jnp.float32),
        ],
    )(node_feat, Wg, a_src, a_dst)


def _sca_body(esed_hbm, src_hbm, dst_hbm, expf_hbm, denp_hbm,
              srcb, dstb, isrc, idst, idxd, esv, edv, pb, zba, gsem, den_sh):
    cid = lax.axis_index("c")
    sid = lax.axis_index("s")
    wid = sid * NC + cid

    def _zero(i, _):
        zba[pl.ds(i * L, L)] = jnp.zeros((L,), jnp.float32)
        return 0
    lax.fori_loop(0, (4 * NPAD // NS) // L, _zero, 0)
    pltpu.sync_copy(zba, den_sh.at[pl.ds(sid * (4 * NPAD // NS),
                                         4 * NPAD // NS)])
    plsc.subcore_barrier()

    def _do_chunk(cno):
        eoff = cno * CHA
        pltpu.sync_copy(src_hbm.at[pl.ds(eoff, CHA)], srcb)
        pltpu.sync_copy(dst_hbm.at[pl.ds(eoff, CHA)], dstb)
        def _bidx(g, _):
            sv = srcb[pl.ds(g * L, L)]
            dv = dstb[pl.ds(g * L, L)]
            for h in range(H):
                isrc[h, pl.ds(g * L, L)] = sv * 8 + h
                idst[h, pl.ds(g * L, L)] = dv * 8 + (4 + h)
                idxd[h, pl.ds(g * L, L)] = dv * 4 + h
            return 0
        lax.fori_loop(0, CHA // L, _bidx, 0)
        cps = []
        for h in range(H):
            cps.append(pltpu.async_copy(esed_hbm.at[isrc.at[h]], esv.at[h], gsem))
            cps.append(pltpu.async_copy(esed_hbm.at[idst.at[h]], edv.at[h], gsem))
        for cp in cps:
            cp.wait()

        def _grp(g, _):
            for h in range(H):
                e = esv[h, pl.ds(g * L, L)] + edv[h, pl.ds(g * L, L)]
                e = jnp.where(e >= 0, e, NEG * e)
                pb[h, pl.ds(g * L, L)] = jnp.exp(e)
            return 0
        lax.fori_loop(0, CHA // L, _grp, 0)
        for h in range(H):
            pltpu.sync_copy(pb.at[h], expf_hbm.at[pl.ds(h * E + eoff, CHA)])
            pltpu.sync_copy(pb.at[h], den_sh.at[idxd.at[h]], add=True)

    def _chunk(j, _):
        _do_chunk(wid + NW * j)
        return 0
    nfull = NCH_A // NW
    lax.fori_loop(0, nfull, _chunk, 0)
    @pl.when(wid < NCH_A - nfull * NW)
    def _():
        _do_chunk(nfull * NW + wid)

    plsc.subcore_barrier()
    w = 4 * NPAD // NS
    pltpu.sync_copy(den_sh.at[pl.ds(sid * w, w)],
                    denp_hbm.at[pl.ds(cid * 4 * NPAD + sid * w, w)])


def _sca(esed_flat, src, dst):
    w = 4 * NPAD // NS
    f = pl.kernel(
        _sca_body,
        out_type=(
            jax.ShapeDtypeStruct((H * E,), jnp.float32),
            jax.ShapeDtypeStruct((NC * 4 * NPAD,), jnp.float32),
        ),
        mesh=_mesh,
        scratch_types=[
            pltpu.VMEM((CHA,), jnp.int32),
            pltpu.VMEM((CHA,), jnp.int32),
            pltpu.VMEM((H, CHA), jnp.int32),
            pltpu.VMEM((H, CHA), jnp.int32),
            pltpu.VMEM((H, CHA), jnp.int32),
            pltpu.VMEM((H, CHA), jnp.float32),
            pltpu.VMEM((H, CHA), jnp.float32),
            pltpu.VMEM((H, CHA), jnp.float32),
            pltpu.VMEM((w,), jnp.float32),
            pltpu.SemaphoreType.DMA,
            pltpu.VMEM_SHARED((4 * NPAD,), jnp.float32),
        ],
    )
    return f(esed_flat, src, dst)


def _sca2_body(expf_hbm, denp_hbm, dst_hbm, alpha_hbm,
               dstb, idxd, pv, d0, d1, gsem):
    cid = lax.axis_index("c")
    sid = lax.axis_index("s")
    wid = sid * NC + cid

    def _do_chunk(cno):
        eoff = cno * CHA
        pltpu.sync_copy(dst_hbm.at[pl.ds(eoff, CHA)], dstb)
        def _bidx(g, _):
            dv = dstb[pl.ds(g * L, L)]
            for h in range(H):
                idxd[h, pl.ds(g * L, L)] = dv * 4 + h
            return 0
        lax.fori_loop(0, CHA // L, _bidx, 0)
        cps = []
        for h in range(H):
            cps.append(pltpu.async_copy(
                expf_hbm.at[pl.ds(h * E + eoff, CHA)], pv.at[h], gsem))
            cps.append(pltpu.async_copy(denp_hbm.at[idxd.at[h]], d0.at[h], gsem))
        for cp in cps:
            cp.wait()
        def _bidx2(g, _):
            for h in range(H):
                idxd[h, pl.ds(g * L, L)] = idxd[h, pl.ds(g * L, L)] + 4 * NPAD
            return 0
        lax.fori_loop(0, CHA // L, _bidx2, 0)
        cps = [pltpu.async_copy(denp_hbm.at[idxd.at[h]], d1.at[h], gsem)
               for h in range(H)]
        for cp in cps:
            cp.wait()
        def _grp(g, _):
            for h in range(H):
                den = d0[h, pl.ds(g * L, L)] + d1[h, pl.ds(g * L, L)]
                pv[h, pl.ds(g * L, L)] = 0.25 * pv[h, pl.ds(g * L, L)] / den
            return 0
        lax.fori_loop(0, CHA // L, _grp, 0)
        for h in range(H):
            pltpu.sync_copy(pv.at[h], alpha_hbm.at[pl.ds(h * E + eoff, CHA)])

    def _chunk(j, _):
        _do_chunk(wid + NW * j)
        return 0
    nfull = NCH_A // NW
    lax.fori_loop(0, nfull, _chunk, 0)
    @pl.when(wid < NCH_A - nfull * NW)
    def _():
        _do_chunk(nfull * NW + wid)


def _sca2(expf, denp, dst):
    f = pl.kernel(
        _sca2_body,
        out_type=jax.ShapeDtypeStruct((H * E,), jnp.float32),
        mesh=_mesh,
        scratch_types=[
            pltpu.VMEM((CHA,), jnp.int32),
            pltpu.VMEM((H, CHA), jnp.int32),
            pltpu.VMEM((H, CHA), jnp.float32),
            pltpu.VMEM((H, CHA), jnp.float32),
            pltpu.VMEM((H, CHA), jnp.float32),
            pltpu.SemaphoreType.DMA,
        ],
    )
    return f(expf, denp, dst)


def _scb_body(zr_hbm, src_hbm, dst_hbm, arep_hbm, hm_hbm,
              srcb2, dsti2, idxg2, rows2, arows2, msg_v, zb,
              lsem, gsem, hacc_sh):
    cid = lax.axis_index("c")
    sid = lax.axis_index("s")
    nb = NBL_B // NS

    def _zb(i, _):
        for j in range(FH // L):
            zb[i, pl.ds(j * L, L)] = jnp.zeros((L,), jnp.float32)
        return 0
    lax.fori_loop(0, 32, _zb, 0)
    for r in range(20):
        pltpu.sync_copy(zb, hacc_sh.at[pl.ds(sid * 640 + r * 32, 32)])
    plsc.subcore_barrier()

    def _lin_cps(jb, p):
        bb = sid + NS * jb
        return (
            pltpu.make_async_copy(src_hbm.at[pl.ds(bb * BCB, BCB)],
                                  srcb2.at[p], lsem),
            pltpu.make_async_copy(dst_hbm.at[pl.ds(bb * BCB, BCB)],
                                  dsti2.at[p], lsem),
            pltpu.make_async_copy(arep_hbm.at[pl.ds(bb * (BCB // 2), BCB // 2)],
                                  arows2.at[p], lsem),
        )

    def fire_lin(jb, p):
        for cp in _lin_cps(jb, p):
            cp.start()

    def wait_lin(jb, p):
        for cp in _lin_cps(jb, p):
            cp.wait()

    def _gat_cp(p):
        return pltpu.make_async_copy(zr_hbm.at[idxg2.at[p]], rows2.at[p], gsem)

    def fire_gather(p):
        def _bidx(g, _):
            sv = srcb2[p, pl.ds(g * L, L)]
            for h in range(H):
                idxg2[p, pl.ds(h * BCB + g * L, L)] = sv * 8 + (h * 2 + cid)
            return 0
        lax.fori_loop(0, BCB // L, _bidx, 0)
        _gat_cp(p).start()

    def compute_scatter(p):
        def _edge(k, _):
            r2 = k // 2
            lo = (k % 2) * 64
            ab0 = arows2[p, r2, pl.ds(lo, L)]
            ab1 = arows2[p, r2, pl.ds(lo + 16, L)]
            ab2 = arows2[p, r2, pl.ds(lo + 32, L)]
            ab3 = arows2[p, r2, pl.ds(lo + 48, L)]
            for j in range(FH // L):
                m = ab0 * rows2[p, k, pl.ds(j * L, L)]
                m = m + ab1 * rows2[p, BCB + k, pl.ds(j * L, L)]
                m = m + ab2 * rows2[p, 2 * BCB + k, pl.ds(j * L, L)]
                m = m + ab3 * rows2[p, 3 * BCB + k, pl.ds(j * L, L)]
                msg_v[k, pl.ds(j * L, L)] = m
            return 0
        lax.fori_loop(0, BCB, _edge, 0)
        pltpu.sync_copy(msg_v, hacc_sh.at[dsti2.at[p]], add=True)

    fire_lin(0, 0)
    wait_lin(0, 0)
    fire_gather(0)
    fire_lin(1, 1)

    def _pair(ji, _):
        jb0 = 2 * ji
        wait_lin(jb0 + 1, 1)
        fire_gather(1)
        _gat_cp(0).wait()
        compute_scatter(0)
        @pl.when(ji < nb // 2 - 1)
        def _():
            fire_lin(jb0 + 2, 0)
        @pl.when(ji < nb // 2 - 1)
        def _():
            wait_lin(jb0 + 2, 0)
            fire_gather(0)
        _gat_cp(1).wait()
        compute_scatter(1)
        @pl.when(ji < nb // 2 - 1)
        def _():
            fire_lin(jb0 + 3, 1)
        return 0
    lax.fori_loop(0, nb // 2, _pair, 0)

    @pl.when(sid < NBL_B - nb * NS)
    def _():
        wait = fire_lin(nb, 0)
        wait_lin(nb, 0)
        fire_gather(0)
        _gat_cp(0).wait()
        compute_scatter(0)

    plsc.subcore_barrier()
    pltpu.sync_copy(hacc_sh.at[pl.ds(sid * 640, 640)],
                    hm_hbm.at[cid, pl.ds(sid * 640, 640)])


def _scb(zr, src, dst, arep):
    f = pl.kernel(
        _scb_body,
        out_type=jax.ShapeDtypeStruct((NC, NPAD, FH), jnp.float32),
        mesh=_mesh,
        scratch_types=[
            pltpu.VMEM((2, BCB), jnp.int32),
            pltpu.VMEM((2, BCB), jnp.int32),
            pltpu.VMEM((2, H * BCB), jnp.int32),
            pltpu.VMEM((2, H * BCB, FH), jnp.float32),
            pltpu.VMEM((2, BCB // 2, FH), jnp.float32),
            pltpu.VMEM((BCB, FH), jnp.float32),
            pltpu.VMEM((32, FH), jnp.float32),
            pltpu.SemaphoreType.DMA,
            pltpu.SemaphoreType.DMA,
            pltpu.VMEM_SHARED((NPAD, FH), jnp.float32),
        ],
    )
    return f(zr, src, dst, arep)


def _tc2_body(hm0_ref, hm1_ref, sim_ref, w_ref, b_ref, wdec_ref, h_ref, g_ref):
    w = w_ref[0]
    acc = jnp.dot(hm0_ref[...], w[:FH, :], preferred_element_type=jnp.float32)
    acc += jnp.dot(hm1_ref[...], w[FH:2 * FH, :], preferred_element_type=jnp.float32)
    acc += jnp.dot(sim_ref[...], w[2 * FH:, :], preferred_element_type=jnp.float32)
    acc += b_ref[0][0:1, :]
    h = jnp.where(acc > 0, acc, jnp.exp(jnp.minimum(acc, 0.0)) - 1.0)
    h_ref[...] = h
    g_ref[...] = jnp.dot(h, wdec_ref[...], preferred_element_type=jnp.float32)


def _tc2(hm0, hm1, sim, w_stack, b_stack, W_dec):
    sel3 = lambda i: (lax.min(i // 4, 1), 0, 0)
    return pl.pallas_call(
        _tc2_body,
        grid=(10,),
        in_specs=[
            pl.BlockSpec((1000, FH), lambda i: (i, 0)),
            pl.BlockSpec((1000, FH), lambda i: (i, 0)),
            pl.BlockSpec((1000, FH), lambda i: (i, 0)),
            pl.BlockSpec((1, 3 * FH, OUT), sel3),
            pl.BlockSpec((1, 8, OUT), sel3),
            pl.BlockSpec((OUT, OUT), lambda i: (0, 0)),
        ],
        out_specs=[
            pl.BlockSpec((1000, OUT), lambda i: (i, 0)),
            pl.BlockSpec((1000, OUT), lambda i: (i, 0)),
        ],
        out_shape=[
            jax.ShapeDtypeStruct((N, OUT), jnp.float32),
            jax.ShapeDtypeStruct((N, OUT), jnp.float32),
        ],
    )(hm0, hm1, sim, w_stack, b_stack, W_dec)


def _scc_body(g_hbm, h_hbm, dis_hbm, mir_hbm, gd_hbm, hm_hbm,
              idxd, idxm, gv, hv, gsem):
    cid = lax.axis_index("c")
    sid = lax.axis_index("s")
    wid = sid * NC + cid
    wbase = wid * PPW

    def _chunk(c, _):
        base = wbase + c * CHC
        pltpu.sync_copy(dis_hbm.at[pl.ds(base, CHC)], idxd)
        pltpu.sync_copy(mir_hbm.at[pl.ds(base, CHC)], idxm)
        cg = pltpu.async_copy(g_hbm.at[idxd], gv, gsem)
        ch = pltpu.async_copy(h_hbm.at[idxm], hv, gsem)
        cg.wait()
        ch.wait()
        pltpu.sync_copy(gv, gd_hbm.at[pl.ds(base, CHC)])
        pltpu.sync_copy(hv, hm_hbm.at[pl.ds(base, CHC)])
        return 0
    lax.fori_loop(0, PPW // CHC, _chunk, 0)


def _scc(G, h, diseases, mrnas):
    f = pl.kernel(
        _scc_body,
        out_type=(
            jax.ShapeDtypeStruct((B, OUT), jnp.float32),
            jax.ShapeDtypeStruct((B, OUT), jnp.float32),
        ),
        mesh=_mesh,
        scratch_types=[
            pltpu.VMEM((CHC,), jnp.int32),
            pltpu.VMEM((CHC,), jnp.int32),
            pltpu.VMEM((CHC, OUT), jnp.float32),
            pltpu.VMEM((CHC, OUT), jnp.float32),
            pltpu.SemaphoreType.DMA,
        ],
    )
    return f(G, h, diseases, mrnas)


def _tc3_body(gd_ref, hm_ref, o_ref):
    s = jnp.sum(gd_ref[...] * hm_ref[...], axis=1)
    o_ref[...] = 1.0 / (1.0 + jnp.exp(-s))


def _tc3(Gd, Hm):
    return pl.pallas_call(
        _tc3_body,
        grid=(8,),
        in_specs=[
            pl.BlockSpec((1024, OUT), lambda i: (i, 0)),
            pl.BlockSpec((1024, OUT), lambda i: (i, 0)),
        ],
        out_specs=pl.BlockSpec((1024,), lambda i: (i,)),
        out_shape=jax.ShapeDtypeStruct((B,), jnp.float32),
    )(Gd, Hm)


def kernel(node_feat, d_sim, m_sim, edge_index, diseases, mrnas,
           Wg, a_src, a_dst, m_fc_W, m_fc_b, d_fc_W, d_fc_b, W_dec):
    src = edge_index[0].astype(jnp.int32)
    dst = edge_index[1].astype(jnp.int32)

    z, esed = _tc1(node_feat, Wg, a_src, a_dst)
    zr = z.reshape(N * 8, FH)

    expf, denp = _sca(esed.reshape(-1), src, dst)
    alphaE = _sca2(expf, denp, dst)

    arep = jnp.broadcast_to(
        alphaE.reshape(H, E).T.reshape(E // 2, 8, 1), (E // 2, 8, L)
    ).reshape(E // 2, 8 * L)

    hm = _scb(zr, src, dst, arep)

    sim = jnp.concatenate([d_sim[:ND], m_sim[ND:]], axis=0)
    w_stack = jnp.stack([d_fc_W, m_fc_W])
    b_stack = jnp.broadcast_to(jnp.stack([d_fc_b, m_fc_b])[:, None, :],
                               (2, 8, OUT))

    h, G = _tc2(hm[0, :N], hm[1, :N], sim, w_stack, b_stack, W_dec)
    Gd, Hm = _scc(G, h, diseases.astype(jnp.int32), mrnas.astype(jnp.int32))
    return _tc3(Gd, Hm)


# speedup vs baseline: 16.6770x; 1.0412x over previous
import jax
import jax.numpy as jnp
from jax import lax
from jax.experimental import pallas as pl
from jax.experimental.pallas import tpu as pltpu
from jax.experimental.pallas import tpu_sc as plsc

N = 10000
E = 160000
H = 4
F = 256
ND = 4000
OUT = 256
B = 8192
NEG = 0.2

NC = 2
NS = 16
L = 16
NW = NC * NS

NPAD = 10240
FH = F // NC

CHA = 128
NCH_A = E // CHA
BCB = 32
NBL_B = E // BCB
PPW = B // NW
CHC = 64

_mesh = plsc.VectorSubcoreMesh(core_axis_name="c", subcore_axis_name="s")


def _tc1_body(x_ref, wg_ref, asrc_ref, adst_ref, z_ref, esed_ref):
    x = x_ref[...]
    wg = wg_ref[...]
    z = jnp.dot(x, wg, preferred_element_type=jnp.float32)
    z_ref[...] = z
    cols = []
    for aref in (asrc_ref, adst_ref):
        for h in range(H):
            a = aref[pl.ds(h, 1), :]
            cols.append(jnp.sum(z[:, h * F:(h + 1) * F] * a, axis=1,
                                keepdims=True))
    esed_ref[...] = jnp.concatenate(cols, axis=1)


def _tc1(node_feat, Wg, a_src, a_dst):
    return pl.pallas_call(
        _tc1_body,
        grid=(10,),
        in_specs=[
            pl.BlockSpec((1000, F), lambda i: (i, 0)),
            pl.BlockSpec((F, H * F), lambda i: (0, 0)),
            pl.BlockSpec((H, F), lambda i: (0, 0)),
            pl.BlockSpec((H, F), lambda i: (0, 0)),
        ],
        out_specs=[
            pl.BlockSpec((1000, H * F), lambda i: (i, 0)),
            pl.BlockSpec((1000, 8), lambda i: (i, 0)),
        ],
        out_shape=[
            jax.ShapeDtypeStruct((N, H * F), jnp.float32),
            jax.ShapeDtypeStruct((N, 8), jnp.float32),
        ],
    )(node_feat, Wg, a_src, a_dst)


def _sca_body(esed_hbm, src_hbm, dst_hbm, expf_hbm, denp_hbm,
              srcb, dstb, isrc, idst, idxd, esv, edv, pb, zba, gsem, den_sh):
    cid = lax.axis_index("c")
    sid = lax.axis_index("s")
    wid = sid * NC + cid

    def _zero(i, _):
        zba[pl.ds(i * L, L)] = jnp.zeros((L,), jnp.float32)
        return 0
    lax.fori_loop(0, (4 * NPAD // NS) // L, _zero, 0)
    pltpu.sync_copy(zba, den_sh.at[pl.ds(sid * (4 * NPAD // NS),
                                         4 * NPAD // NS)])
    plsc.subcore_barrier()

    def _do_chunk(cno):
        eoff = cno * CHA
        pltpu.sync_copy(src_hbm.at[pl.ds(eoff, CHA)], srcb)
        pltpu.sync_copy(dst_hbm.at[pl.ds(eoff, CHA)], dstb)
        def _bidx(g, _):
            sv = srcb[pl.ds(g * L, L)]
            dv = dstb[pl.ds(g * L, L)]
            for h in range(H):
                isrc[h, pl.ds(g * L, L)] = sv * 8 + h
                idst[h, pl.ds(g * L, L)] = dv * 8 + (4 + h)
                idxd[h, pl.ds(g * L, L)] = dv * 4 + h
            return 0
        lax.fori_loop(0, CHA // L, _bidx, 0)
        cps = []
        for h in range(H):
            cps.append(pltpu.async_copy(esed_hbm.at[isrc.at[h]], esv.at[h], gsem))
            cps.append(pltpu.async_copy(esed_hbm.at[idst.at[h]], edv.at[h], gsem))
        for cp in cps:
            cp.wait()

        def _grp(g, _):
            for h in range(H):
                e = esv[h, pl.ds(g * L, L)] + edv[h, pl.ds(g * L, L)]
                e = jnp.where(e >= 0, e, NEG * e)
                pb[h, pl.ds(g * L, L)] = jnp.exp(e)
            return 0
        lax.fori_loop(0, CHA // L, _grp, 0)
        for h in range(H):
            pltpu.sync_copy(pb.at[h], expf_hbm.at[pl.ds(h * E + eoff, CHA)])
            pltpu.sync_copy(pb.at[h], den_sh.at[idxd.at[h]], add=True)

    def _chunk(j, _):
        _do_chunk(wid + NW * j)
        return 0
    nfull = NCH_A // NW
    lax.fori_loop(0, nfull, _chunk, 0)
    @pl.when(wid < NCH_A - nfull * NW)
    def _():
        _do_chunk(nfull * NW + wid)

    plsc.subcore_barrier()
    w = 4 * NPAD // NS
    pltpu.sync_copy(den_sh.at[pl.ds(sid * w, w)],
                    denp_hbm.at[pl.ds(cid * 4 * NPAD + sid * w, w)])


def _sca(esed_flat, src, dst):
    w = 4 * NPAD // NS
    f = pl.kernel(
        _sca_body,
        out_type=(
            jax.ShapeDtypeStruct((H * E,), jnp.float32),
            jax.ShapeDtypeStruct((NC * 4 * NPAD,), jnp.float32),
        ),
        mesh=_mesh,
        scratch_types=[
            pltpu.VMEM((CHA,), jnp.int32),
            pltpu.VMEM((CHA,), jnp.int32),
            pltpu.VMEM((H, CHA), jnp.int32),
            pltpu.VMEM((H, CHA), jnp.int32),
            pltpu.VMEM((H, CHA), jnp.int32),
            pltpu.VMEM((H, CHA), jnp.float32),
            pltpu.VMEM((H, CHA), jnp.float32),
            pltpu.VMEM((H, CHA), jnp.float32),
            pltpu.VMEM((w,), jnp.float32),
            pltpu.SemaphoreType.DMA,
            pltpu.VMEM_SHARED((4 * NPAD,), jnp.float32),
        ],
    )
    return f(esed_flat, src, dst)


def _sca2_body(expf_hbm, denp_hbm, dst_hbm, alpha_hbm,
               dstb, idxd, pv, d0, d1, gsem):
    cid = lax.axis_index("c")
    sid = lax.axis_index("s")
    wid = sid * NC + cid

    def _do_chunk(cno):
        eoff = cno * CHA
        pltpu.sync_copy(dst_hbm.at[pl.ds(eoff, CHA)], dstb)
        def _bidx(g, _):
            dv = dstb[pl.ds(g * L, L)]
            for h in range(H):
                idxd[h, pl.ds(g * L, L)] = dv * 4 + h
            return 0
        lax.fori_loop(0, CHA // L, _bidx, 0)
        cps = []
        for h in range(H):
            cps.append(pltpu.async_copy(
                expf_hbm.at[pl.ds(h * E + eoff, CHA)], pv.at[h], gsem))
            cps.append(pltpu.async_copy(denp_hbm.at[idxd.at[h]], d0.at[h], gsem))
        for cp in cps:
            cp.wait()
        def _bidx2(g, _):
            for h in range(H):
                idxd[h, pl.ds(g * L, L)] = idxd[h, pl.ds(g * L, L)] + 4 * NPAD
            return 0
        lax.fori_loop(0, CHA // L, _bidx2, 0)
        cps = [pltpu.async_copy(denp_hbm.at[idxd.at[h]], d1.at[h], gsem)
               for h in range(H)]
        for cp in cps:
            cp.wait()
        def _grp(g, _):
            for h in range(H):
                den = d0[h, pl.ds(g * L, L)] + d1[h, pl.ds(g * L, L)]
                pv[h, pl.ds(g * L, L)] = 0.25 * pv[h, pl.ds(g * L, L)] / den
            return 0
        lax.fori_loop(0, CHA // L, _grp, 0)
        for h in range(H):
            pltpu.sync_copy(pv.at[h], alpha_hbm.at[pl.ds(h * E + eoff, CHA)])

    def _chunk(j, _):
        _do_chunk(wid + NW * j)
        return 0
    nfull = NCH_A // NW
    lax.fori_loop(0, nfull, _chunk, 0)
    @pl.when(wid < NCH_A - nfull * NW)
    def _():
        _do_chunk(nfull * NW + wid)


def _sca2(expf, denp, dst):
    f = pl.kernel(
        _sca2_body,
        out_type=jax.ShapeDtypeStruct((H * E,), jnp.float32),
        mesh=_mesh,
        scratch_types=[
            pltpu.VMEM((CHA,), jnp.int32),
            pltpu.VMEM((H, CHA), jnp.int32),
            pltpu.VMEM((H, CHA), jnp.float32),
            pltpu.VMEM((H, CHA), jnp.float32),
            pltpu.VMEM((H, CHA), jnp.float32),
            pltpu.SemaphoreType.DMA,
        ],
    )
    return f(expf, denp, dst)


def _scb_body(zr_hbm, src_hbm, dst_hbm, arep_hbm, hm_hbm,
              srcb2, dsti2, dscat2, idxg2, rows2, arows2, msg2, zb,
              lsem, gsem, ssem, hacc_sh):
    cid = lax.axis_index("c")
    sid = lax.axis_index("s")
    nb = NBL_B // NS

    def _zb(i, _):
        for j in range(FH // L):
            zb[i, pl.ds(j * L, L)] = jnp.zeros((L,), jnp.float32)
        return 0
    lax.fori_loop(0, 16, _zb, 0)
    for r in range(40):
        pltpu.sync_copy(zb, hacc_sh.at[pl.ds(sid * 640 + r * 16, 16)])
    plsc.subcore_barrier()

    def _lin_cps(jb, p):
        bb = sid + NS * jb
        return (
            pltpu.make_async_copy(src_hbm.at[pl.ds(bb * BCB, BCB)],
                                  srcb2.at[p], lsem),
            pltpu.make_async_copy(dst_hbm.at[pl.ds(bb * BCB, BCB)],
                                  dsti2.at[p], lsem),
            pltpu.make_async_copy(arep_hbm.at[pl.ds(bb * (BCB // 2), BCB // 2)],
                                  arows2.at[p], lsem),
        )

    def fire_lin(jb, p):
        for cp in _lin_cps(jb, p):
            cp.start()

    def wait_lin(jb, p):
        for cp in _lin_cps(jb, p):
            cp.wait()

    def _gat_cp(p):
        return pltpu.make_async_copy(zr_hbm.at[idxg2.at[p]], rows2.at[p], gsem)

    def fire_gather(p):
        def _bidx(g, _):
            sv = srcb2[p, pl.ds(g * L, L)]
            for h in range(H):
                idxg2[p, pl.ds(h * BCB + g * L, L)] = sv * 8 + (h * 2 + cid)
            return 0
        lax.fori_loop(0, BCB // L, _bidx, 0)
        _gat_cp(p).start()

    def _sct_cp(p):
        return pltpu.make_async_copy(msg2.at[p], hacc_sh.at[dscat2.at[p]], ssem)

    def compute_scatter(p):
        def _edge(k, _):
            r2 = k // 2
            lo = (k % 2) * 64
            ab0 = arows2[p, r2, pl.ds(lo, L)]
            ab1 = arows2[p, r2, pl.ds(lo + 16, L)]
            ab2 = arows2[p, r2, pl.ds(lo + 32, L)]
            ab3 = arows2[p, r2, pl.ds(lo + 48, L)]
            for j in range(FH // L):
                m = ab0 * rows2[p, k, pl.ds(j * L, L)]
                m = m + ab1 * rows2[p, BCB + k, pl.ds(j * L, L)]
                m = m + ab2 * rows2[p, 2 * BCB + k, pl.ds(j * L, L)]
                m = m + ab3 * rows2[p, 3 * BCB + k, pl.ds(j * L, L)]
                msg2[p, k, pl.ds(j * L, L)] = m
            return 0
        lax.fori_loop(0, BCB, _edge, 0)
        for g in range(BCB // L):
            dscat2[p, pl.ds(g * L, L)] = dsti2[p, pl.ds(g * L, L)]
        _sct_cp(p).start()

    fire_lin(0, 0)
    wait_lin(0, 0)
    fire_gather(0)
    fire_lin(1, 1)

    def _pair(ji, _):
        jb0 = 2 * ji
        wait_lin(jb0 + 1, 1)
        fire_gather(1)
        _gat_cp(0).wait()
        @pl.when(ji > 0)
        def _():
            _sct_cp(0).wait()
        compute_scatter(0)
        @pl.when(ji < nb // 2 - 1)
        def _():
            fire_lin(jb0 + 2, 0)
        @pl.when(ji < nb // 2 - 1)
        def _():
            wait_lin(jb0 + 2, 0)
            fire_gather(0)
        _gat_cp(1).wait()
        @pl.when(ji > 0)
        def _():
            _sct_cp(1).wait()
        compute_scatter(1)
        @pl.when(ji < nb // 2 - 1)
        def _():
            fire_lin(jb0 + 3, 1)
        return 0
    lax.fori_loop(0, nb // 2, _pair, 0)
    _sct_cp(1).wait()

    @pl.when(sid < NBL_B - nb * NS)
    def _():
        _sct_cp(0).wait()
        fire_lin(nb, 0)
        wait_lin(nb, 0)
        fire_gather(0)
        _gat_cp(0).wait()
        compute_scatter(0)
        _sct_cp(0).wait()
    @pl.when(sid >= NBL_B - nb * NS)
    def _():
        _sct_cp(0).wait()

    plsc.subcore_barrier()
    pltpu.sync_copy(hacc_sh.at[pl.ds(sid * 640, 640)],
                    hm_hbm.at[cid, pl.ds(sid * 640, 640)])


def _scb(zr, src, dst, arep):
    f = pl.kernel(
        _scb_body,
        out_type=jax.ShapeDtypeStruct((NC, NPAD, FH), jnp.float32),
        mesh=_mesh,
        scratch_types=[
            pltpu.VMEM((2, BCB), jnp.int32),
            pltpu.VMEM((2, BCB), jnp.int32),
            pltpu.VMEM((2, BCB), jnp.int32),
            pltpu.VMEM((2, H * BCB), jnp.int32),
            pltpu.VMEM((2, H * BCB, FH), jnp.float32),
            pltpu.VMEM((2, BCB // 2, FH), jnp.float32),
            pltpu.VMEM((2, BCB, FH), jnp.float32),
            pltpu.VMEM((16, FH), jnp.float32),
            pltpu.SemaphoreType.DMA,
            pltpu.SemaphoreType.DMA,
            pltpu.SemaphoreType.DMA,
            pltpu.VMEM_SHARED((NPAD, FH), jnp.float32),
        ],
    )
    return f(zr, src, dst, arep)


def _tc2_body(hm0_ref, hm1_ref, sim_ref, w_ref, b_ref, wdec_ref, h_ref, g_ref):
    w = w_ref[0]
    acc = jnp.dot(hm0_ref[...], w[:FH, :], preferred_element_type=jnp.float32)
    acc += jnp.dot(hm1_ref[...], w[FH:2 * FH, :], preferred_element_type=jnp.float32)
    acc += jnp.dot(sim_ref[...], w[2 * FH:, :], preferred_element_type=jnp.float32)
    acc += b_ref[0][0:1, :]
    h = jnp.where(acc > 0, acc, jnp.exp(jnp.minimum(acc, 0.0)) - 1.0)
    h_ref[...] = h
    g_ref[...] = jnp.dot(h, wdec_ref[...], preferred_element_type=jnp.float32)


def _tc2(hm0, hm1, sim, w_stack, b_stack, W_dec):
    sel3 = lambda i: (lax.min(i // 4, 1), 0, 0)
    return pl.pallas_call(
        _tc2_body,
        grid=(10,),
        in_specs=[
            pl.BlockSpec((1000, FH), lambda i: (i, 0)),
            pl.BlockSpec((1000, FH), lambda i: (i, 0)),
            pl.BlockSpec((1000, FH), lambda i: (i, 0)),
            pl.BlockSpec((1, 3 * FH, OUT), sel3),
            pl.BlockSpec((1, 8, OUT), sel3),
            pl.BlockSpec((OUT, OUT), lambda i: (0, 0)),
        ],
        out_specs=[
            pl.BlockSpec((1000, OUT), lambda i: (i, 0)),
            pl.BlockSpec((1000, OUT), lambda i: (i, 0)),
        ],
        out_shape=[
            jax.ShapeDtypeStruct((N, OUT), jnp.float32),
            jax.ShapeDtypeStruct((N, OUT), jnp.float32),
        ],
    )(hm0, hm1, sim, w_stack, b_stack, W_dec)


def _scc_body(g_hbm, h_hbm, dis_hbm, mir_hbm, gd_hbm, hm_hbm,
              idxd, idxm, gv, hv, gsem):
    cid = lax.axis_index("c")
    sid = lax.axis_index("s")
    wid = sid * NC + cid
    wbase = wid * PPW

    def _chunk(c, _):
        base = wbase + c * CHC
        pltpu.sync_copy(dis_hbm.at[pl.ds(base, CHC)], idxd)
        pltpu.sync_copy(mir_hbm.at[pl.ds(base, CHC)], idxm)
        cg = pltpu.async_copy(g_hbm.at[idxd], gv, gsem)
        ch = pltpu.async_copy(h_hbm.at[idxm], hv, gsem)
        cg.wait()
        ch.wait()
        pltpu.sync_copy(gv, gd_hbm.at[pl.ds(base, CHC)])
        pltpu.sync_copy(hv, hm_hbm.at[pl.ds(base, CHC)])
        return 0
    lax.fori_loop(0, PPW // CHC, _chunk, 0)


def _scc(G, h, diseases, mrnas):
    f = pl.kernel(
        _scc_body,
        out_type=(
            jax.ShapeDtypeStruct((B, OUT), jnp.float32),
            jax.ShapeDtypeStruct((B, OUT), jnp.float32),
        ),
        mesh=_mesh,
        scratch_types=[
            pltpu.VMEM((CHC,), jnp.int32),
            pltpu.VMEM((CHC,), jnp.int32),
            pltpu.VMEM((CHC, OUT), jnp.float32),
            pltpu.VMEM((CHC, OUT), jnp.float32),
            pltpu.SemaphoreType.DMA,
        ],
    )
    return f(G, h, diseases, mrnas)


def _tc3_body(gd_ref, hm_ref, o_ref):
    s = jnp.sum(gd_ref[...] * hm_ref[...], axis=1)
    o_ref[...] = 1.0 / (1.0 + jnp.exp(-s))


def _tc3(Gd, Hm):
    return pl.pallas_call(
        _tc3_body,
        grid=(8,),
        in_specs=[
            pl.BlockSpec((1024, OUT), lambda i: (i, 0)),
            pl.BlockSpec((1024, OUT), lambda i: (i, 0)),
        ],
        out_specs=pl.BlockSpec((1024,), lambda i: (i,)),
        out_shape=jax.ShapeDtypeStruct((B,), jnp.float32),
    )(Gd, Hm)


def kernel(node_feat, d_sim, m_sim, edge_index, diseases, mrnas,
           Wg, a_src, a_dst, m_fc_W, m_fc_b, d_fc_W, d_fc_b, W_dec):
    src = edge_index[0].astype(jnp.int32)
    dst = edge_index[1].astype(jnp.int32)

    z, esed = _tc1(node_feat, Wg, a_src, a_dst)
    zr = z.reshape(N * 8, FH)

    expf, denp = _sca(esed.reshape(-1), src, dst)
    alphaE = _sca2(expf, denp, dst)

    arep = jnp.broadcast_to(
        alphaE.reshape(H, E).T.reshape(E // 2, 8, 1), (E // 2, 8, L)
    ).reshape(E // 2, 8 * L)

    hm = _scb(zr, src, dst, arep)

    sim = jnp.concatenate([d_sim[:ND], m_sim[ND:]], axis=0)
    w_stack = jnp.stack([d_fc_W, m_fc_W])
    b_stack = jnp.broadcast_to(jnp.stack([d_fc_b, m_fc_b])[:, None, :],
                               (2, 8, OUT))

    h, G = _tc2(hm[0, :N], hm[1, :N], sim, w_stack, b_stack, W_dec)
    Gd, Hm = _scc(G, h, diseases.astype(jnp.int32), mrnas.astype(jnp.int32))
    return _tc3(Gd, Hm)
